# Initial kernel scaffold; baseline (speedup 1.0000x reference)
import jax
import jax.numpy as jnp
from jax.experimental import pallas as pl


def kernel(x, W_E):
    raise NotImplementedError("write your pallas kernel here")


# speedup vs baseline: 7.4489x; 7.4489x over previous
import functools

import jax
import jax.numpy as jnp
from jax import lax
from jax.experimental import pallas as pl
from jax.experimental.pallas import tpu as pltpu
from jax.experimental.pallas import tpu_sc as plsc

D_MODEL = 128
NC, NS = 2, 16
NW = NC * NS
SUB = 128
GROUP = 512


def _embed_gather(table, x2d, n_tokens):
    per_w = n_tokens // NW
    n_groups = per_w // GROUP
    rows_per_group = GROUP // SUB

    mesh = plsc.VectorSubcoreMesh(core_axis_name="c", subcore_axis_name="s")

    @functools.partial(
        pl.kernel,
        mesh=mesh,
        out_type=jax.ShapeDtypeStruct((n_tokens, D_MODEL), jnp.float32),
        scratch_types=[
            pltpu.VMEM((rows_per_group, SUB), jnp.int32),
            pltpu.VMEM((GROUP, D_MODEL), jnp.float32),
            pltpu.SemaphoreType.DMA,
        ],
    )
    def k(x_hbm, tab_hbm, out_hbm, idx_v, rows_v, sem):
        wid = lax.axis_index("s") * NC + lax.axis_index("c")

        def body(g, carry):
            blk = wid * n_groups + g
            base = blk * GROUP
            row0 = blk * rows_per_group
            pltpu.sync_copy(x_hbm.at[pl.ds(row0, rows_per_group)], idx_v)
            copies = [
                pltpu.async_copy(
                    tab_hbm.at[idx_v.at[j]],
                    rows_v.at[pl.ds(j * SUB, SUB)],
                    sem,
                )
                for j in range(rows_per_group)
            ]
            for c in copies:
                c.wait()
            pltpu.sync_copy(rows_v, out_hbm.at[pl.ds(base, GROUP)])
            return carry

        lax.fori_loop(0, n_groups, body, 0)

    return k(x2d, table)


def kernel(x, W_E):
    batch, pos = x.shape
    n_tokens = batch * pos
    x2d = x.reshape(n_tokens // SUB, SUB).astype(jnp.int32)
    table = W_E.T
    out = _embed_gather(table, x2d, n_tokens)
    return out.reshape(batch, pos, D_MODEL)


# speedup vs baseline: 8.4854x; 1.1392x over previous
import functools

import jax
import jax.numpy as jnp
from jax import lax
from jax.experimental import pallas as pl
from jax.experimental.pallas import tpu as pltpu
from jax.experimental.pallas import tpu_sc as plsc

D_MODEL = 128
NC, NS = 2, 16
NW = NC * NS
GROUP = 128
RING = 5


def _embed_gather(table, x_flat, n_tokens):
    per_w = n_tokens // NW
    n_groups = per_w // GROUP
    t_main = n_groups // RING - 1

    mesh = plsc.VectorSubcoreMesh(core_axis_name="c", subcore_axis_name="s")

    @functools.partial(
        pl.kernel,
        mesh=mesh,
        out_type=jax.ShapeDtypeStruct((n_tokens, D_MODEL), jnp.float32),
        scratch_types=[
            pltpu.VMEM((per_w,), jnp.int32),
            pltpu.VMEM((RING, GROUP, D_MODEL), jnp.float32),
        ]
        + [pltpu.SemaphoreType.DMA] * RING
        + [pltpu.SemaphoreType.DMA],
    )
    def k(x_hbm, tab_hbm, out_hbm, idx_v, rows_v, *sems):
        gsems, ssem = sems[:RING], sems[RING]
        wid = lax.axis_index("s") * NC + lax.axis_index("c")
        ibase = wid * per_w

        pltpu.sync_copy(x_hbm.at[pl.ds(ibase, per_w)], idx_v)

        def fire_gather(g, r):
            pltpu.async_copy(
                tab_hbm.at[idx_v.at[pl.ds(g * GROUP, GROUP)]],
                rows_v.at[r],
                gsems[r],
            )

        def wait_gather(r):
            pltpu.make_async_copy(
                tab_hbm.at[pl.ds(0, GROUP)], rows_v.at[r], gsems[r]
            ).wait()

        def store(g, r):
            pltpu.async_copy(
                rows_v.at[r], out_hbm.at[pl.ds(ibase + g * GROUP, GROUP)], ssem
            ).wait()

        for r in range(RING):
            fire_gather(r, r)

        def body(t, carry):
            for r in range(RING):
                g = t * RING + r
                wait_gather(r)
                store(g, r)
                fire_gather(g + RING, r)
            return carry

        lax.fori_loop(0, t_main, body, 0)

        for r in range(RING):
            g = n_groups - RING + r
            wait_gather(r)
            store(g, r)

    return k(x_flat, table)


def kernel(x, W_E):
    batch, pos = x.shape
    n_tokens = batch * pos
    x_flat = x.reshape(n_tokens).astype(jnp.int32)
    table = W_E.T
    out = _embed_gather(table, x_flat, n_tokens)
    return out.reshape(batch, pos, D_MODEL)
